# baseline (speedup 1.0000x reference)
import functools

import jax
import jax.numpy as jnp
from jax import lax
from jax.experimental import pallas as pl
from jax.experimental.pallas import tpu as pltpu
from jax.experimental.pallas import tpu_sc as plsc

VOCAB = 1000000
BATCH = 4096
SEQ = 50
DIM = 64
NC, NS = 2, 16
NW = NC * NS
BPW = BATCH // NW
BB = 8
RND = BPW // BB
NBUF = 4

TRB = 8192
NTRB = -(-VOCAB // TRB)

_mesh = plsc.VectorSubcoreMesh(core_axis_name="c", subcore_axis_name="s")


def _transpose_body(x_ref, o_ref, t_ref):
    t_ref[...] = x_ref[...].T
    o_ref[:, 0:DIM] = t_ref[0::2, :]
    o_ref[:, DIM:] = t_ref[1::2, :]


_linearize = pl.pallas_call(
    _transpose_body,
    grid=(NTRB,),
    in_specs=[pl.BlockSpec((DIM, TRB), lambda i: (0, i))],
    out_specs=pl.BlockSpec((TRB // 2, 2 * DIM), lambda i: (i, 0)),
    out_shape=jax.ShapeDtypeStruct((VOCAB // 2, 2 * DIM), jnp.float32),
    scratch_shapes=[pltpu.VMEM((TRB, DIM), jnp.float32)],
)


@functools.partial(
    pl.kernel,
    mesh=_mesh,
    compiler_params=pltpu.CompilerParams(use_tc_tiling_on_sc=False),
    out_type=jax.ShapeDtypeStruct((BATCH, SEQ, DIM), jnp.float32),
    scratch_types=[
        pltpu.VMEM((BPW, SEQ), jnp.int32),
        pltpu.VMEM((NBUF, BB, SEQ, DIM), jnp.float32),
        [pltpu.SemaphoreType.DMA] * NBUF,
        [pltpu.SemaphoreType.DMA] * NBUF,
    ],
)
def _gather_kernel(idx_hbm, table_hbm, out_hbm, idx_v, rows_v, sem_g, sem_w):
    wid = lax.axis_index("s") * NC + lax.axis_index("c")
    base = wid * BPW
    pltpu.sync_copy(idx_hbm.at[pl.ds(base, BPW)], idx_v)

    def fire(r, j):
        for q in range(BB):
            pltpu.async_copy(
                table_hbm.at[idx_v.at[r * BB + q]],
                rows_v.at[j, q],
                sem_g[j],
            )

    def drain_gathers(j):
        for q in range(BB):
            pltpu.make_async_copy(
                table_hbm.at[idx_v.at[0]],
                rows_v.at[j, q],
                sem_g[j],
            ).wait()

    def wait_write(j):
        pltpu.make_async_copy(
            rows_v.at[j],
            out_hbm.at[pl.ds(base, BB)],
            sem_w[j],
        ).wait()

    for j in range(NBUF):
        fire(j, j)

    def outer(t, carry):
        for j in range(NBUF):
            r = t * NBUF + j
            drain_gathers(j)
            pltpu.async_copy(
                rows_v.at[j],
                out_hbm.at[pl.ds(base + r * BB, BB)],
                sem_w[j],
            )

            @pl.when(r + NBUF < RND)
            def _():
                wait_write(j)
                fire(r + NBUF, j)

        return carry

    lax.fori_loop(0, RND // NBUF, outer, 0)
    for j in range(NBUF):
        wait_write(j)


def kernel(input_ids, weight):
    wlin = _linearize(weight.T)
    return _gather_kernel(
        input_ids.astype(jnp.int32), wlin.reshape(VOCAB, DIM)
    )


# speedup vs baseline: 1.0322x; 1.0322x over previous
import functools

import jax
import jax.numpy as jnp
from jax import lax
from jax.experimental import pallas as pl
from jax.experimental.pallas import tpu as pltpu
from jax.experimental.pallas import tpu_sc as plsc

VOCAB = 1000000
BATCH = 4096
SEQ = 50
DIM = 64
NC, NS = 2, 16
NW = NC * NS
BPW = BATCH // NW
BB = 8
RND = BPW // BB
NBUF = 4

TRB = 16384
NTRB = -(-VOCAB // TRB)

_mesh = plsc.VectorSubcoreMesh(core_axis_name="c", subcore_axis_name="s")


def _transpose_body(x_ref, o_ref, t_ref):
    t_ref[...] = x_ref[...].T
    o_ref[:, 0:DIM] = t_ref[0::2, :]
    o_ref[:, DIM:] = t_ref[1::2, :]


_linearize = pl.pallas_call(
    _transpose_body,
    grid=(NTRB,),
    in_specs=[pl.BlockSpec((DIM, TRB), lambda i: (0, i))],
    out_specs=pl.BlockSpec((TRB // 2, 2 * DIM), lambda i: (i, 0)),
    out_shape=jax.ShapeDtypeStruct((VOCAB // 2, 2 * DIM), jnp.float32),
    scratch_shapes=[pltpu.VMEM((TRB, DIM), jnp.float32)],
)


@functools.partial(
    pl.kernel,
    mesh=_mesh,
    compiler_params=pltpu.CompilerParams(use_tc_tiling_on_sc=False),
    out_type=jax.ShapeDtypeStruct((BATCH, SEQ, DIM), jnp.float32),
    scratch_types=[
        pltpu.VMEM((BPW, SEQ), jnp.int32),
        pltpu.VMEM((NBUF, BB, SEQ, DIM), jnp.float32),
        [pltpu.SemaphoreType.DMA] * NBUF,
        [pltpu.SemaphoreType.DMA] * NBUF,
    ],
)
def _gather_kernel(idx_hbm, table_hbm, out_hbm, idx_v, rows_v, sem_g, sem_w):
    wid = lax.axis_index("s") * NC + lax.axis_index("c")
    base = wid * BPW
    pltpu.sync_copy(idx_hbm.at[pl.ds(base, BPW)], idx_v)

    def fire(r, j):
        for q in range(BB):
            pltpu.async_copy(
                table_hbm.at[idx_v.at[r * BB + q]],
                rows_v.at[j, q],
                sem_g[j],
            )

    def drain_gathers(j):
        for q in range(BB):
            pltpu.make_async_copy(
                table_hbm.at[idx_v.at[0]],
                rows_v.at[j, q],
                sem_g[j],
            ).wait()

    def wait_write(j):
        pltpu.make_async_copy(
            rows_v.at[j],
            out_hbm.at[pl.ds(base, BB)],
            sem_w[j],
        ).wait()

    for j in range(NBUF):
        fire(j, j)

    def outer(t, carry):
        for j in range(NBUF):
            r = t * NBUF + j
            drain_gathers(j)
            pltpu.async_copy(
                rows_v.at[j],
                out_hbm.at[pl.ds(base + r * BB, BB)],
                sem_w[j],
            )

            @pl.when(r + NBUF < RND)
            def _():
                wait_write(j)
                fire(r + NBUF, j)

        return carry

    lax.fori_loop(0, RND // NBUF, outer, 0)
    for j in range(NBUF):
        wait_write(j)


def kernel(input_ids, weight):
    wlin = _linearize(weight.T)
    return _gather_kernel(
        input_ids.astype(jnp.int32), wlin.reshape(VOCAB, DIM)
    )
